# baseline (speedup 1.0000x reference)
import functools

import jax
import jax.numpy as jnp
from jax import lax
from jax.experimental import pallas as pl
from jax.experimental.pallas import tpu as pltpu
from jax.experimental.pallas import tpu_sc as plsc

HIDDEN = 768
EPS = 1e-12

_NCORES = 2
_NSUB = 16
_NW = _NCORES * _NSUB
_CHUNK = 32
_NBUF = 4

_TOK = 512


def _sc_gather(word_emb, ids):
    n = ids.shape[0]
    per_w = n // _NW
    nchunk = per_w // _CHUNK
    mesh = plsc.VectorSubcoreMesh(core_axis_name="core",
                                  subcore_axis_name="subcore")

    @functools.partial(
        pl.kernel, mesh=mesh,
        out_type=jax.ShapeDtypeStruct((n, HIDDEN), word_emb.dtype),
        scratch_types=(
            [pltpu.VMEM((per_w,), jnp.int32)]
            + [pltpu.VMEM((_CHUNK, HIDDEN), jnp.float32)] * _NBUF
            + [pltpu.SemaphoreType.DMA] * (1 + 2 * _NBUF)
        ),
    )
    def gather_kernel(table_hbm, idx_hbm, o_hbm, idx_v, *rest):
        bufs = rest[:_NBUF]
        isem = rest[_NBUF]
        gsems = rest[_NBUF + 1:2 * _NBUF + 1]
        wsems = rest[2 * _NBUF + 1:]
        wid = lax.axis_index("subcore") * _NCORES + lax.axis_index("core")
        base = wid * per_w
        pltpu.async_copy(idx_hbm.at[pl.ds(base, per_w)], idx_v, isem).wait()

        def start_gather(j, b):
            pltpu.async_copy(
                table_hbm.at[idx_v.at[pl.ds(j * _CHUNK, _CHUNK)]],
                bufs[b], gsems[b])

        def wait_gather(j, b):
            pltpu.make_async_copy(
                table_hbm.at[idx_v.at[pl.ds(j * _CHUNK, _CHUNK)]],
                bufs[b], gsems[b]).wait()

        def start_write(j, b):
            pltpu.async_copy(
                bufs[b], o_hbm.at[pl.ds(base + j * _CHUNK, _CHUNK)], wsems[b])

        def wait_write(j, b):
            pltpu.make_async_copy(
                bufs[b], o_hbm.at[pl.ds(base + j * _CHUNK, _CHUNK)], wsems[b]
            ).wait()

        for b in range(_NBUF):
            start_gather(b, b)

        @pl.loop(0, nchunk // _NBUF)
        def _(p):
            j0 = p * _NBUF
            for b in range(_NBUF):
                j = j0 + b

                @pl.when(p > 0)
                def _():
                    wait_write(j - _NBUF, b)

                wait_gather(j, b)
                start_write(j, b)

                @pl.when(p < nchunk // _NBUF - 1)
                def _():
                    start_gather(j + _NBUF, b)

        for b in range(_NBUF):
            wait_write(nchunk - _NBUF + b, b)

    return gather_kernel(word_emb, ids)


def _ln_body(g_ref, tt_ref, pos_ref, type_ref, gamma_ref, beta_ref, o_ref):
    x = g_ref[...]
    x = x + pos_ref[...]
    ttf = tt_ref[...]
    t0 = type_ref[0:1, :]
    t1 = type_ref[1:2, :]
    x = x + t0 + ttf * (t1 - t0)
    mean = jnp.mean(x, axis=-1, keepdims=True)
    xc = x - mean
    var = jnp.mean(xc * xc, axis=-1, keepdims=True)
    y = xc * jax.lax.rsqrt(var + EPS)
    o_ref[...] = y * gamma_ref[...] + beta_ref[...]


def _ln_body_acc(g_ref, tt_ref, pos_ref, type_ref, gamma_ref, beta_ref,
                 acc_ref, o_ref):
    del acc_ref
    _ln_body(g_ref, tt_ref, pos_ref, type_ref, gamma_ref, beta_ref, o_ref)


def _tc_layernorm_chunk(gathered, ttf, pos_emb, type_emb, gamma, beta,
                        acc, chunk, n_chunks):
    n = acc.shape[0]
    nk = n // n_chunks
    blk0 = chunk * (nk // _TOK)
    return pl.pallas_call(
        _ln_body_acc,
        grid=(nk // _TOK,),
        in_specs=[
            pl.BlockSpec((_TOK, HIDDEN), lambda i: (i, 0)),
            pl.BlockSpec((_TOK, 1), lambda i: (i, 0)),
            pl.BlockSpec((_TOK, HIDDEN), lambda i: (0, 0)),
            pl.BlockSpec((2, HIDDEN), lambda i: (0, 0)),
            pl.BlockSpec((1, HIDDEN), lambda i: (0, 0)),
            pl.BlockSpec((1, HIDDEN), lambda i: (0, 0)),
            pl.BlockSpec(memory_space=pl.ANY),
        ],
        out_specs=pl.BlockSpec((_TOK, HIDDEN), lambda i: (blk0 + i, 0)),
        out_shape=jax.ShapeDtypeStruct((n, HIDDEN), jnp.float32),
        input_output_aliases={6: 0},
    )(gathered, ttf, pos_emb, type_emb, gamma, beta, acc)


def _tc_layernorm_first(gathered, ttf, pos_emb, type_emb, gamma, beta,
                        n, n_chunks):
    nk = n // n_chunks
    return pl.pallas_call(
        _ln_body,
        grid=(nk // _TOK,),
        in_specs=[
            pl.BlockSpec((_TOK, HIDDEN), lambda i: (i, 0)),
            pl.BlockSpec((_TOK, 1), lambda i: (i, 0)),
            pl.BlockSpec((_TOK, HIDDEN), lambda i: (0, 0)),
            pl.BlockSpec((2, HIDDEN), lambda i: (0, 0)),
            pl.BlockSpec((1, HIDDEN), lambda i: (0, 0)),
            pl.BlockSpec((1, HIDDEN), lambda i: (0, 0)),
        ],
        out_specs=pl.BlockSpec((_TOK, HIDDEN), lambda i: (i, 0)),
        out_shape=jax.ShapeDtypeStruct((n, HIDDEN), jnp.float32),
    )(gathered, ttf, pos_emb, type_emb, gamma, beta)


_NCHUNKS = 1


def kernel(input_ids, token_type_ids, word_emb, type_emb, pos_emb,
           ln_gamma, ln_beta):
    b, s = input_ids.shape
    n = b * s
    nk = n // _NCHUNKS
    ids_flat = input_ids.reshape(n).astype(jnp.int32)
    ttf = token_type_ids.reshape(n, 1).astype(jnp.float32)
    gamma = ln_gamma.reshape(1, HIDDEN)
    beta = ln_beta.reshape(1, HIDDEN)

    gathered = [
        _sc_gather(word_emb, lax.dynamic_slice_in_dim(ids_flat, k * nk, nk))
        for k in range(_NCHUNKS)
    ]
    acc = _tc_layernorm_first(gathered[0], ttf[0:nk], pos_emb, type_emb,
                              gamma, beta, n, _NCHUNKS)
    for k in range(1, _NCHUNKS):
        acc = _tc_layernorm_chunk(gathered[k], ttf[k * nk:(k + 1) * nk],
                                  pos_emb, type_emb, gamma, beta,
                                  acc, k, _NCHUNKS)
    return acc.reshape(b, s, HIDDEN)


# speedup vs baseline: 1.0971x; 1.0971x over previous
import functools

import jax
import jax.numpy as jnp
from jax import lax
from jax.experimental import pallas as pl
from jax.experimental.pallas import tpu as pltpu
from jax.experimental.pallas import tpu_sc as plsc

HIDDEN = 768
EPS = 1e-12

_NCORES = 2
_NSUB = 16
_NW = _NCORES * _NSUB
_CHUNK = 64

_TOK = 1024


def _sc_gather(word_emb, ids):
    n = ids.shape[0]
    per_w = n // _NW
    nchunk = per_w // _CHUNK
    mesh = plsc.VectorSubcoreMesh(core_axis_name="core",
                                  subcore_axis_name="subcore")

    @functools.partial(
        pl.kernel, mesh=mesh,
        out_type=jax.ShapeDtypeStruct((n, HIDDEN), word_emb.dtype),
        scratch_types=[
            pltpu.VMEM((per_w,), jnp.int32),
            pltpu.VMEM((_CHUNK, HIDDEN), jnp.float32),
            pltpu.VMEM((_CHUNK, HIDDEN), jnp.float32),
            pltpu.SemaphoreType.DMA,
            pltpu.SemaphoreType.DMA,
            pltpu.SemaphoreType.DMA,
            pltpu.SemaphoreType.DMA,
            pltpu.SemaphoreType.DMA,
        ],
    )
    def gather_kernel(table_hbm, idx_hbm, o_hbm, idx_v, b0, b1,
                      isem, g0, g1, w0, w1):
        wid = lax.axis_index("subcore") * _NCORES + lax.axis_index("core")
        base = wid * per_w
        pltpu.async_copy(idx_hbm.at[pl.ds(base, per_w)], idx_v, isem).wait()

        def start_gather(j, buf, sem):
            pltpu.async_copy(
                table_hbm.at[idx_v.at[pl.ds(j * _CHUNK, _CHUNK)]], buf, sem)

        def wait_gather(j, buf, sem):
            pltpu.make_async_copy(
                table_hbm.at[idx_v.at[pl.ds(j * _CHUNK, _CHUNK)]], buf, sem
            ).wait()

        def start_write(j, buf, sem):
            pltpu.async_copy(
                buf, o_hbm.at[pl.ds(base + j * _CHUNK, _CHUNK)], sem)

        def wait_write(j, buf, sem):
            pltpu.make_async_copy(
                buf, o_hbm.at[pl.ds(base + j * _CHUNK, _CHUNK)], sem
            ).wait()

        start_gather(0, b0, g0)

        @pl.loop(0, nchunk // 2)
        def _(p):
            j = 2 * p

            @pl.when(p > 0)
            def _():
                wait_write(j - 1, b1, w1)

            start_gather(j + 1, b1, g1)
            wait_gather(j, b0, g0)
            start_write(j, b0, w0)

            @pl.when(p < nchunk // 2 - 1)
            def _():
                wait_write(j, b0, w0)
                start_gather(j + 2, b0, g0)

            wait_gather(j + 1, b1, g1)
            start_write(j + 1, b1, w1)

        wait_write(nchunk - 2, b0, w0)
        wait_write(nchunk - 1, b1, w1)

    return gather_kernel(word_emb, ids)


def _ln_body(g_ref, tt_ref, pos_ref, type_ref, gamma_ref, beta_ref, o_ref):
    x = g_ref[...]
    x = x + pos_ref[...]
    ttf = tt_ref[...]
    t0 = type_ref[0:1, :]
    t1 = type_ref[1:2, :]
    x = x + t0 + ttf * (t1 - t0)
    mean = jnp.mean(x, axis=-1, keepdims=True)
    xc = x - mean
    var = jnp.mean(xc * xc, axis=-1, keepdims=True)
    y = xc * jax.lax.rsqrt(var + EPS)
    o_ref[...] = y * gamma_ref[...] + beta_ref[...]


def _tc_layernorm(gathered, ttf, pos_emb, type_emb, gamma, beta):
    n = gathered.shape[0]
    return pl.pallas_call(
        _ln_body,
        grid=(n // _TOK,),
        in_specs=[
            pl.BlockSpec((_TOK, HIDDEN), lambda i: (i, 0)),
            pl.BlockSpec((_TOK, 1), lambda i: (i, 0)),
            pl.BlockSpec((_TOK, HIDDEN), lambda i: (0, 0)),
            pl.BlockSpec((2, HIDDEN), lambda i: (0, 0)),
            pl.BlockSpec((1, HIDDEN), lambda i: (0, 0)),
            pl.BlockSpec((1, HIDDEN), lambda i: (0, 0)),
        ],
        out_specs=pl.BlockSpec((_TOK, HIDDEN), lambda i: (i, 0)),
        out_shape=jax.ShapeDtypeStruct((n, HIDDEN), jnp.float32),
    )(gathered, ttf, pos_emb, type_emb, gamma, beta)


def kernel(input_ids, token_type_ids, word_emb, type_emb, pos_emb,
           ln_gamma, ln_beta):
    b, s = input_ids.shape
    n = b * s
    ids_flat = input_ids.reshape(n).astype(jnp.int32)
    gathered = _sc_gather(word_emb, ids_flat)
    ttf = token_type_ids.reshape(n, 1).astype(jnp.float32)
    out = _tc_layernorm(
        gathered,
        ttf,
        jnp.tile(pos_emb, (_TOK // s, 1)) if _TOK > s else pos_emb,
        type_emb,
        ln_gamma.reshape(1, HIDDEN),
        ln_beta.reshape(1, HIDDEN),
    )
    return out.reshape(b, s, HIDDEN)


# speedup vs baseline: 1.1408x; 1.0399x over previous
import functools

import jax
import jax.numpy as jnp
from jax import lax
from jax.experimental import pallas as pl
from jax.experimental.pallas import tpu as pltpu
from jax.experimental.pallas import tpu_sc as plsc

HIDDEN = 768
EPS = 1e-12

_NCORES = 2
_NSUB = 16
_NW = _NCORES * _NSUB
_CHUNK = 64

_TOK = 2048


def _sc_gather(word_emb, ids):
    n = ids.shape[0]
    per_w = n // _NW
    nchunk = per_w // _CHUNK
    mesh = plsc.VectorSubcoreMesh(core_axis_name="core",
                                  subcore_axis_name="subcore")

    @functools.partial(
        pl.kernel, mesh=mesh,
        out_type=jax.ShapeDtypeStruct((n, HIDDEN), word_emb.dtype),
        scratch_types=[
            pltpu.VMEM((per_w,), jnp.int32),
            pltpu.VMEM((_CHUNK, HIDDEN), jnp.float32),
            pltpu.VMEM((_CHUNK, HIDDEN), jnp.float32),
            pltpu.SemaphoreType.DMA,
            pltpu.SemaphoreType.DMA,
            pltpu.SemaphoreType.DMA,
            pltpu.SemaphoreType.DMA,
            pltpu.SemaphoreType.DMA,
        ],
    )
    def gather_kernel(table_hbm, idx_hbm, o_hbm, idx_v, b0, b1,
                      isem, g0, g1, w0, w1):
        wid = lax.axis_index("subcore") * _NCORES + lax.axis_index("core")
        base = wid * per_w
        pltpu.async_copy(idx_hbm.at[pl.ds(base, per_w)], idx_v, isem).wait()

        def start_gather(j, buf, sem):
            pltpu.async_copy(
                table_hbm.at[idx_v.at[pl.ds(j * _CHUNK, _CHUNK)]], buf, sem)

        def wait_gather(j, buf, sem):
            pltpu.make_async_copy(
                table_hbm.at[idx_v.at[pl.ds(j * _CHUNK, _CHUNK)]], buf, sem
            ).wait()

        def start_write(j, buf, sem):
            pltpu.async_copy(
                buf, o_hbm.at[pl.ds(base + j * _CHUNK, _CHUNK)], sem)

        def wait_write(j, buf, sem):
            pltpu.make_async_copy(
                buf, o_hbm.at[pl.ds(base + j * _CHUNK, _CHUNK)], sem
            ).wait()

        start_gather(0, b0, g0)

        @pl.loop(0, nchunk // 2)
        def _(p):
            j = 2 * p

            @pl.when(p > 0)
            def _():
                wait_write(j - 1, b1, w1)

            start_gather(j + 1, b1, g1)
            wait_gather(j, b0, g0)
            start_write(j, b0, w0)

            @pl.when(p < nchunk // 2 - 1)
            def _():
                wait_write(j, b0, w0)
                start_gather(j + 2, b0, g0)

            wait_gather(j + 1, b1, g1)
            start_write(j + 1, b1, w1)

        wait_write(nchunk - 2, b0, w0)
        wait_write(nchunk - 1, b1, w1)

    return gather_kernel(word_emb, ids)


def _ln_body(g_ref, tt_ref, pos_ref, type_ref, gamma_ref, beta_ref, o_ref):
    x = g_ref[...]
    x = x + pos_ref[...]
    ttf = tt_ref[...]
    t0 = type_ref[0:1, :]
    t1 = type_ref[1:2, :]
    x = x + t0 + ttf * (t1 - t0)
    mean = jnp.mean(x, axis=-1, keepdims=True)
    xc = x - mean
    var = jnp.mean(xc * xc, axis=-1, keepdims=True)
    y = xc * jax.lax.rsqrt(var + EPS)
    o_ref[...] = y * gamma_ref[...] + beta_ref[...]


def _tc_layernorm(gathered, ttf, pos_emb, type_emb, gamma, beta):
    n = gathered.shape[0]
    return pl.pallas_call(
        _ln_body,
        grid=(n // _TOK,),
        in_specs=[
            pl.BlockSpec((_TOK, HIDDEN), lambda i: (i, 0)),
            pl.BlockSpec((_TOK, 1), lambda i: (i, 0)),
            pl.BlockSpec((_TOK, HIDDEN), lambda i: (0, 0)),
            pl.BlockSpec((2, HIDDEN), lambda i: (0, 0)),
            pl.BlockSpec((1, HIDDEN), lambda i: (0, 0)),
            pl.BlockSpec((1, HIDDEN), lambda i: (0, 0)),
        ],
        out_specs=pl.BlockSpec((_TOK, HIDDEN), lambda i: (i, 0)),
        out_shape=jax.ShapeDtypeStruct((n, HIDDEN), jnp.float32),
    )(gathered, ttf, pos_emb, type_emb, gamma, beta)


def kernel(input_ids, token_type_ids, word_emb, type_emb, pos_emb,
           ln_gamma, ln_beta):
    b, s = input_ids.shape
    n = b * s
    ids_flat = input_ids.reshape(n).astype(jnp.int32)
    gathered = _sc_gather(word_emb, ids_flat)
    ttf = token_type_ids.reshape(n, 1).astype(jnp.float32)
    out = _tc_layernorm(
        gathered,
        ttf,
        jnp.tile(pos_emb, (_TOK // s, 1)) if _TOK > s else pos_emb,
        type_emb,
        ln_gamma.reshape(1, HIDDEN),
        ln_beta.reshape(1, HIDDEN),
    )
    return out.reshape(b, s, HIDDEN)


# speedup vs baseline: 1.1418x; 1.0009x over previous
import functools

import jax
import jax.numpy as jnp
from jax import lax
from jax.experimental import pallas as pl
from jax.experimental.pallas import tpu as pltpu
from jax.experimental.pallas import tpu_sc as plsc

HIDDEN = 768
EPS = 1e-12

_NCORES = 2
_NSUB = 16
_NW = _NCORES * _NSUB
_CHUNK = 64

_TOK = 2048


def _sc_gather(word_emb, ids):
    n = ids.shape[0]
    per_w = n // _NW
    nchunk = per_w // _CHUNK
    mesh = plsc.VectorSubcoreMesh(core_axis_name="core",
                                  subcore_axis_name="subcore")

    @functools.partial(
        pl.kernel, mesh=mesh,
        out_type=jax.ShapeDtypeStruct((n, HIDDEN), word_emb.dtype),
        scratch_types=[
            pltpu.VMEM((per_w,), jnp.int32),
            pltpu.VMEM((_CHUNK, HIDDEN), jnp.float32),
            pltpu.VMEM((_CHUNK, HIDDEN), jnp.float32),
            pltpu.SemaphoreType.DMA,
            pltpu.SemaphoreType.DMA,
            pltpu.SemaphoreType.DMA,
            pltpu.SemaphoreType.DMA,
            pltpu.SemaphoreType.DMA,
        ],
    )
    def gather_kernel(table_hbm, idx_hbm, o_hbm, idx_v, b0, b1,
                      isem, g0, g1, w0, w1):
        wid = lax.axis_index("subcore") * _NCORES + lax.axis_index("core")
        base = wid * per_w
        pltpu.async_copy(idx_hbm.at[pl.ds(base, per_w)], idx_v, isem).wait()

        def start_gather(j, buf, sem):
            pltpu.async_copy(
                table_hbm.at[idx_v.at[pl.ds(j * _CHUNK, _CHUNK)]], buf, sem)

        def wait_gather(j, buf, sem):
            pltpu.make_async_copy(
                table_hbm.at[idx_v.at[pl.ds(j * _CHUNK, _CHUNK)]], buf, sem
            ).wait()

        def start_write(j, buf, sem):
            pltpu.async_copy(
                buf, o_hbm.at[pl.ds(base + j * _CHUNK, _CHUNK)], sem)

        def wait_write(j, buf, sem):
            pltpu.make_async_copy(
                buf, o_hbm.at[pl.ds(base + j * _CHUNK, _CHUNK)], sem
            ).wait()

        start_gather(0, b0, g0)

        @pl.loop(0, nchunk // 2)
        def _(p):
            j = 2 * p

            @pl.when(p > 0)
            def _():
                wait_write(j - 1, b1, w1)

            start_gather(j + 1, b1, g1)
            wait_gather(j, b0, g0)
            start_write(j, b0, w0)

            @pl.when(p < nchunk // 2 - 1)
            def _():
                wait_write(j, b0, w0)
                start_gather(j + 2, b0, g0)

            wait_gather(j + 1, b1, g1)
            start_write(j + 1, b1, w1)

        wait_write(nchunk - 2, b0, w0)
        wait_write(nchunk - 1, b1, w1)

    return gather_kernel(word_emb, ids)


def _ln_body(g_ref, tt_ref, pos_ref, type_ref, gamma_ref, beta_ref, o_ref):
    x = g_ref[...]
    x = x + pos_ref[...]
    ttf = tt_ref[...]
    t0 = type_ref[0:1, :]
    t1 = type_ref[1:2, :]
    x = x + t0 + ttf * (t1 - t0)
    mean = jnp.mean(x, axis=-1, keepdims=True)
    xc = x - mean
    var = jnp.mean(xc * xc, axis=-1, keepdims=True)
    y = xc * jax.lax.rsqrt(var + EPS)
    o_ref[...] = y * gamma_ref[...] + beta_ref[...]


def _ln_body_acc(g_ref, tt_ref, pos_ref, type_ref, gamma_ref, beta_ref,
                 acc_ref, o_ref):
    del acc_ref
    _ln_body(g_ref, tt_ref, pos_ref, type_ref, gamma_ref, beta_ref, o_ref)


def _tc_layernorm_chunk(gathered, ttf, pos_emb, type_emb, gamma, beta,
                        n, chunk, n_chunks, acc):
    nk = n // n_chunks
    blk0 = chunk * (nk // _TOK)
    body = _ln_body if acc is None else _ln_body_acc
    in_specs = [
        pl.BlockSpec((_TOK, HIDDEN), lambda i: (i, 0)),
        pl.BlockSpec((_TOK, 1), lambda i: (blk0 + i, 0)),
        pl.BlockSpec((_TOK, HIDDEN), lambda i: (0, 0)),
        pl.BlockSpec((2, HIDDEN), lambda i: (0, 0)),
        pl.BlockSpec((1, HIDDEN), lambda i: (0, 0)),
        pl.BlockSpec((1, HIDDEN), lambda i: (0, 0)),
    ]
    args = [gathered, ttf, pos_emb, type_emb, gamma, beta]
    aliases = {}
    if acc is not None:
        in_specs.append(pl.BlockSpec(memory_space=pl.ANY))
        args.append(acc)
        aliases = {6: 0}
    return pl.pallas_call(
        body,
        grid=(nk // _TOK,),
        in_specs=in_specs,
        out_specs=pl.BlockSpec((_TOK, HIDDEN), lambda i: (blk0 + i, 0)),
        out_shape=jax.ShapeDtypeStruct((n, HIDDEN), jnp.float32),
        input_output_aliases=aliases,
    )(*args)


_NCHUNKS = 2


def kernel(input_ids, token_type_ids, word_emb, type_emb, pos_emb,
           ln_gamma, ln_beta):
    b, s = input_ids.shape
    n = b * s
    nk = n // _NCHUNKS
    ids_flat = input_ids.reshape(n).astype(jnp.int32)
    ttf = token_type_ids.reshape(n, 1).astype(jnp.float32)
    gamma = ln_gamma.reshape(1, HIDDEN)
    beta = ln_beta.reshape(1, HIDDEN)
    pos = jnp.tile(pos_emb, (_TOK // s, 1)) if _TOK > s else pos_emb

    gathered = [
        _sc_gather(word_emb, lax.dynamic_slice_in_dim(ids_flat, k * nk, nk))
        for k in range(_NCHUNKS)
    ]
    acc = None
    for k in range(_NCHUNKS):
        acc = _tc_layernorm_chunk(gathered[k], ttf, pos, type_emb,
                                  gamma, beta, n, k, _NCHUNKS, acc)
    return acc.reshape(b, s, HIDDEN)
